# baseline (speedup 1.0000x reference)
import functools

import jax
import jax.numpy as jnp
from jax import lax
from jax.experimental import pallas as pl
from jax.experimental.pallas import tpu as pltpu

N = 10000
NFEAT = 128
NHID = 64
NCLASS = 32
ALPHA = 0.2
BM1 = 320
BM2 = 200


LOG2E = 1.4426950408889634


def _prep_kernel(x_ref, w_ref, asrc_ref, adstT_ref,
                 whe_ref, whebf_ref, fc_ref, fdst_ref, fd2_ref, maxd_ref,
                 meanwh_ref):
    wh = jnp.dot(x_ref[...], w_ref[...], preferred_element_type=jnp.float32)
    whe = jnp.concatenate(
        [wh, jnp.ones((wh.shape[0], 1), jnp.float32)], axis=1)
    whe_ref[...] = whe
    whebf_ref[...] = whe.astype(jnp.bfloat16)
    fc_ref[...] = jnp.dot(wh, asrc_ref[...], preferred_element_type=jnp.float32)
    fdst = lax.dot_general(adstT_ref[...], wh, (((1,), (1,)), ((), ())),
                           preferred_element_type=jnp.float32)
    fdst_ref[...] = fdst
    fd2_ref[...] = ALPHA * fdst
    maxd_ref[...] = jnp.max(fdst, keepdims=True).reshape(1, 1)
    meanwh_ref[...] = jnp.mean(wh, axis=0, keepdims=True)


def _prep(x, w, a, out_dim):
    asrc = LOG2E * a[:out_dim]
    adstT = LOG2E * a[out_dim:].T
    n, _ = x.shape
    return pl.pallas_call(
        _prep_kernel,
        out_shape=(
            jax.ShapeDtypeStruct((n, out_dim + 1), jnp.float32),
            jax.ShapeDtypeStruct((n, out_dim + 1), jnp.bfloat16),
            jax.ShapeDtypeStruct((n, 1), jnp.float32),
            jax.ShapeDtypeStruct((1, n), jnp.float32),
            jax.ShapeDtypeStruct((1, n), jnp.float32),
            jax.ShapeDtypeStruct((1, 1), jnp.float32),
            jax.ShapeDtypeStruct((1, out_dim), jnp.float32),
        ),
    )(x, w, asrc, adstT)


def _rowcoef(fc_ref, maxd_ref):
    fc = fc_ref[...]
    u = fc + maxd_ref[...]
    m = jnp.maximum(u, ALPHA * u)
    return fc - m, ALPHA * fc - m


def _layer1_kernel(adj_ref, whe_ref, fc_ref, maxd_ref, fdst_ref, fd2_ref,
                   meanwh_ref, out_ref, minadj_ref, *, out_dim):
    adjv = adj_ref[...]
    a, b = _rowcoef(fc_ref, maxd_ref)
    t = jnp.maximum(a + fdst_ref[...], b + fd2_ref[...])
    p = jnp.where(adjv > 0, jnp.exp2(t), 0.0)
    minadj_ref[...] = jnp.min(adjv, axis=1, keepdims=True)
    oe = jnp.dot(p, whe_ref[...], preferred_element_type=jnp.float32)
    denom = oe[:, out_dim:out_dim + 1]
    o = oe[:, :out_dim]
    o = jnp.where(denom > 0, o / denom, meanwh_ref[...])
    o = jnp.where(o > 0, o, jnp.exp(o) - 1.0)
    out_ref[...] = o


def _layer1(adj, prepped, out_dim):
    whe, whebf, fc, fdst, fd2, maxd, meanwh = prepped
    n = adj.shape[0]
    return pl.pallas_call(
        functools.partial(_layer1_kernel, out_dim=out_dim),
        grid=(pl.cdiv(n, BM1),),
        in_specs=[
            pl.BlockSpec((BM1, n), lambda i: (i, 0)),
            pl.BlockSpec((n, out_dim + 1), lambda i: (0, 0)),
            pl.BlockSpec((BM1, 1), lambda i: (i, 0)),
            pl.BlockSpec((1, 1), lambda i: (0, 0)),
            pl.BlockSpec((1, n), lambda i: (0, 0)),
            pl.BlockSpec((1, n), lambda i: (0, 0)),
            pl.BlockSpec((1, out_dim), lambda i: (0, 0)),
        ],
        out_specs=(
            pl.BlockSpec((BM1, out_dim), lambda i: (i, 0)),
            pl.BlockSpec((BM1, 1), lambda i: (i, 0)),
        ),
        out_shape=(
            jax.ShapeDtypeStruct((n, out_dim), jnp.float32),
            jax.ShapeDtypeStruct((n, 1), jnp.float32),
        ),
    )(adj, whe, fc, maxd, fdst, fd2, meanwh)


def _layer2_kernel(adjidx_ref, zflags_ref, adj_ref, whe_ref, fc_ref, maxd_ref,
                   fdst_ref, fd2_ref, meanwh_ref, out_ref, *, out_dim):
    i = pl.program_id(0)
    flagged = zflags_ref[i] != 0
    a, b = _rowcoef(fc_ref, maxd_ref)

    @pl.when(jnp.logical_not(flagged))
    def _fast():
        t = jnp.maximum(a + fdst_ref[...], b + fd2_ref[...])
        p = jnp.exp2(t).astype(jnp.bfloat16)
        oe = jnp.dot(p, whe_ref[...], preferred_element_type=jnp.float32)
        denom = oe[:, out_dim:out_dim + 1]
        o = oe[:, :out_dim]
        out_ref[...] = jnp.where(denom > 0, o / denom, meanwh_ref[...])

    @pl.when(flagged)
    def _slow():
        t = jnp.maximum(a + fdst_ref[...], b + fd2_ref[...])
        p = jnp.where(adj_ref[...] > 0, jnp.exp2(t), 0.0).astype(jnp.bfloat16)
        oe = jnp.dot(p, whe_ref[...], preferred_element_type=jnp.float32)
        denom = oe[:, out_dim:out_dim + 1]
        o = oe[:, :out_dim]
        out_ref[...] = jnp.where(denom > 0, o / denom, meanwh_ref[...])


def _layer2(adjidx, zflags, adj, prepped, out_dim):
    whe, whebf, fc, fdst, fd2, maxd, meanwh = prepped
    whe = whebf
    n = adj.shape[0]
    grid_spec = pltpu.PrefetchScalarGridSpec(
        num_scalar_prefetch=2,
        grid=(n // BM2,),
        in_specs=[
            pl.BlockSpec((BM2, n), lambda i, ai, zf: (ai[i], 0)),
            pl.BlockSpec((n, out_dim + 1), lambda i, ai, zf: (0, 0)),
            pl.BlockSpec((BM2, 1), lambda i, ai, zf: (i, 0)),
            pl.BlockSpec((1, 1), lambda i, ai, zf: (0, 0)),
            pl.BlockSpec((1, n), lambda i, ai, zf: (0, 0)),
            pl.BlockSpec((1, n), lambda i, ai, zf: (0, 0)),
            pl.BlockSpec((1, out_dim), lambda i, ai, zf: (0, 0)),
        ],
        out_specs=pl.BlockSpec((BM2, out_dim), lambda i, ai, zf: (i, 0)),
    )
    return pl.pallas_call(
        functools.partial(_layer2_kernel, out_dim=out_dim),
        grid_spec=grid_spec,
        out_shape=jax.ShapeDtypeStruct((n, out_dim), jnp.float32),
    )(adjidx, zflags, adj, whe, fc, maxd, fdst, fd2, meanwh)


@jax.jit
def kernel(x, adj, W1, a1, W2, a2):
    prepped1 = _prep(x, W1, a1, NHID)
    h, minadj = _layer1(adj, prepped1, NHID)
    zflags = (minadj.reshape(N // BM2, BM2).min(axis=1) <= 0).astype(jnp.int32)
    g = N // BM2
    idx = jnp.arange(g, dtype=jnp.int32)
    adjidx = jax.lax.cummax(jnp.where(zflags != 0, idx, -1))
    adjidx = jnp.where(adjidx < 0, 0, adjidx)
    prepped2 = _prep(h, W2, a2, NCLASS)
    return _layer2(adjidx, zflags, adj, prepped2, NCLASS)


# speedup vs baseline: 1.0502x; 1.0502x over previous
import functools

import jax
import jax.numpy as jnp
from jax import lax
from jax.experimental import pallas as pl
from jax.experimental.pallas import tpu as pltpu

N = 10000
NFEAT = 128
NHID = 64
NCLASS = 32
ALPHA = 0.2
BM1 = 320
BM2 = 200


LOG2E = 1.4426950408889634


def _prep_kernel(x_ref, w_ref, asrc_ref, adstT_ref,
                 whe_ref, fc_ref, fdst_ref, fd2_ref, maxd_ref, meanwh_ref):
    wh = jnp.dot(x_ref[...], w_ref[...], preferred_element_type=jnp.float32)
    whe_ref[...] = jnp.concatenate(
        [wh, jnp.ones((wh.shape[0], 1), jnp.float32)], axis=1)
    fc_ref[...] = jnp.dot(wh, asrc_ref[...], preferred_element_type=jnp.float32)
    fdst = lax.dot_general(adstT_ref[...], wh, (((1,), (1,)), ((), ())),
                           preferred_element_type=jnp.float32)
    fdst_ref[...] = fdst
    fd2_ref[...] = ALPHA * fdst
    maxd_ref[...] = jnp.max(fdst, keepdims=True).reshape(1, 1)
    meanwh_ref[...] = jnp.mean(wh, axis=0, keepdims=True)


def _prep(x, w, a, out_dim):
    asrc = LOG2E * a[:out_dim]
    adstT = LOG2E * a[out_dim:].T
    n, _ = x.shape
    return pl.pallas_call(
        _prep_kernel,
        out_shape=(
            jax.ShapeDtypeStruct((n, out_dim + 1), jnp.float32),
            jax.ShapeDtypeStruct((n, 1), jnp.float32),
            jax.ShapeDtypeStruct((1, n), jnp.float32),
            jax.ShapeDtypeStruct((1, n), jnp.float32),
            jax.ShapeDtypeStruct((1, 1), jnp.float32),
            jax.ShapeDtypeStruct((1, out_dim), jnp.float32),
        ),
    )(x, w, asrc, adstT)


def _rowcoef(fc_ref, maxd_ref):
    fc = fc_ref[...]
    u = fc + maxd_ref[...]
    m = jnp.maximum(u, ALPHA * u)
    return fc - m, ALPHA * fc - m


def _layer1_kernel(adj_ref, whe_ref, fc_ref, maxd_ref, fdst_ref, fd2_ref,
                   meanwh_ref, out_ref, minadj_ref, *, out_dim):
    adjv = adj_ref[...]
    a, b = _rowcoef(fc_ref, maxd_ref)
    t = jnp.maximum(a + fdst_ref[...], b + fd2_ref[...])
    p = jnp.where(adjv > 0, jnp.exp2(t), 0.0)
    minadj_ref[...] = jnp.min(adjv, axis=1, keepdims=True)
    oe = jnp.dot(p, whe_ref[...], preferred_element_type=jnp.float32)
    denom = oe[:, out_dim:out_dim + 1]
    o = oe[:, :out_dim]
    o = jnp.where(denom > 0, o / denom, meanwh_ref[...])
    o = jnp.where(o > 0, o, jnp.exp(o) - 1.0)
    out_ref[...] = o


def _layer1(adj, prepped, out_dim):
    whe, fc, fdst, fd2, maxd, meanwh = prepped
    n = adj.shape[0]
    return pl.pallas_call(
        functools.partial(_layer1_kernel, out_dim=out_dim),
        grid=(pl.cdiv(n, BM1),),
        in_specs=[
            pl.BlockSpec((BM1, n), lambda i: (i, 0)),
            pl.BlockSpec((n, out_dim + 1), lambda i: (0, 0)),
            pl.BlockSpec((BM1, 1), lambda i: (i, 0)),
            pl.BlockSpec((1, 1), lambda i: (0, 0)),
            pl.BlockSpec((1, n), lambda i: (0, 0)),
            pl.BlockSpec((1, n), lambda i: (0, 0)),
            pl.BlockSpec((1, out_dim), lambda i: (0, 0)),
        ],
        out_specs=(
            pl.BlockSpec((BM1, out_dim), lambda i: (i, 0)),
            pl.BlockSpec((BM1, 1), lambda i: (i, 0)),
        ),
        out_shape=(
            jax.ShapeDtypeStruct((n, out_dim), jnp.float32),
            jax.ShapeDtypeStruct((n, 1), jnp.float32),
        ),
    )(adj, whe, fc, maxd, fdst, fd2, meanwh)


def _layer2_kernel(adjidx_ref, zflags_ref, adj_ref, whe_ref, fc_ref, maxd_ref,
                   fdst_ref, fd2_ref, meanwh_ref, out_ref, *, out_dim):
    i = pl.program_id(0)
    flagged = zflags_ref[i] != 0
    a, b = _rowcoef(fc_ref, maxd_ref)

    @pl.when(jnp.logical_not(flagged))
    def _fast():
        t = jnp.maximum(a + fdst_ref[...], b + fd2_ref[...])
        p = jnp.exp2(t)
        oe = jnp.dot(p, whe_ref[...], preferred_element_type=jnp.float32)
        denom = oe[:, out_dim:out_dim + 1]
        o = oe[:, :out_dim]
        out_ref[...] = jnp.where(denom > 0, o / denom, meanwh_ref[...])

    @pl.when(flagged)
    def _slow():
        t = jnp.maximum(a + fdst_ref[...], b + fd2_ref[...])
        p = jnp.where(adj_ref[...] > 0, jnp.exp2(t), 0.0)
        oe = jnp.dot(p, whe_ref[...], preferred_element_type=jnp.float32)
        denom = oe[:, out_dim:out_dim + 1]
        o = oe[:, :out_dim]
        out_ref[...] = jnp.where(denom > 0, o / denom, meanwh_ref[...])


def _layer2(adjidx, zflags, adj, prepped, out_dim):
    whe, fc, fdst, fd2, maxd, meanwh = prepped
    n = adj.shape[0]
    grid_spec = pltpu.PrefetchScalarGridSpec(
        num_scalar_prefetch=2,
        grid=(n // BM2,),
        in_specs=[
            pl.BlockSpec((BM2, n), lambda i, ai, zf: (ai[i], 0)),
            pl.BlockSpec((n, out_dim + 1), lambda i, ai, zf: (0, 0)),
            pl.BlockSpec((BM2, 1), lambda i, ai, zf: (i, 0)),
            pl.BlockSpec((1, 1), lambda i, ai, zf: (0, 0)),
            pl.BlockSpec((1, n), lambda i, ai, zf: (0, 0)),
            pl.BlockSpec((1, n), lambda i, ai, zf: (0, 0)),
            pl.BlockSpec((1, out_dim), lambda i, ai, zf: (0, 0)),
        ],
        out_specs=pl.BlockSpec((BM2, out_dim), lambda i, ai, zf: (i, 0)),
    )
    return pl.pallas_call(
        functools.partial(_layer2_kernel, out_dim=out_dim),
        grid_spec=grid_spec,
        out_shape=jax.ShapeDtypeStruct((n, out_dim), jnp.float32),
    )(adjidx, zflags, adj, whe, fc, maxd, fdst, fd2, meanwh)


@jax.jit
def kernel(x, adj, W1, a1, W2, a2):
    prepped1 = _prep(x, W1, a1, NHID)
    h, minadj = _layer1(adj, prepped1, NHID)
    zflags = (minadj.reshape(N // BM2, BM2).min(axis=1) <= 0).astype(jnp.int32)
    g = N // BM2
    idx = jnp.arange(g, dtype=jnp.int32)
    adjidx = jax.lax.cummax(jnp.where(zflags != 0, idx, -1))
    adjidx = jnp.where(adjidx < 0, 0, adjidx)
    prepped2 = _prep(h, W2, a2, NCLASS)
    return _layer2(adjidx, zflags, adj, prepped2, NCLASS)
